# baseline (speedup 1.0000x reference)
import functools

import jax
import jax.numpy as jnp
from jax import lax
from jax.experimental import pallas as pl
from jax.experimental.pallas import tpu as pltpu
from jax.experimental.pallas import tpu_sc as plsc

EMBEDDING_DIM = 64
BATCH = 16384

_INFO = plsc.get_sparse_core_info()
_NC = _INFO.num_cores
_NS = _INFO.num_subcores
_NW = _NC * _NS
_B_PER_W = BATCH // _NW
_CHUNK = 128
_NCHUNK = _B_PER_W // _CHUNK

_mesh = plsc.VectorSubcoreMesh(core_axis_name="c", subcore_axis_name="s")


@functools.partial(
    pl.kernel,
    mesh=_mesh,
    compiler_params=pltpu.CompilerParams(use_tc_tiling_on_sc=False),
    out_type=[
        jax.ShapeDtypeStruct((BATCH, EMBEDDING_DIM), jnp.float32),
        jax.ShapeDtypeStruct((BATCH, EMBEDDING_DIM), jnp.float32),
    ],
    scratch_types=[
        pltpu.VMEM((_NCHUNK, _CHUNK), jnp.int32),
        pltpu.VMEM((_B_PER_W, EMBEDDING_DIM), jnp.float32),
        pltpu.VMEM((_B_PER_W, EMBEDDING_DIM), jnp.float32),
        pltpu.SemaphoreType.DMA,
        pltpu.SemaphoreType.DMA,
    ],
)
def _dual_gather(idx_hbm, timbre_hbm, speaker_hbm, out_t_hbm, out_s_hbm,
                 idx_v, rows_t, rows_s, sem_t, sem_s):
    wid = lax.axis_index("s") * _NC + lax.axis_index("c")
    base = wid * _B_PER_W
    pltpu.sync_copy(idx_hbm.at[pl.ds(wid * _NCHUNK, _NCHUNK)], idx_v)
    copies = []
    for j in range(_NCHUNK):
        rsl = pl.ds(j * _CHUNK, _CHUNK)
        copies.append(
            pltpu.async_copy(timbre_hbm.at[idx_v.at[j]], rows_t.at[rsl], sem_t))
        copies.append(
            pltpu.async_copy(speaker_hbm.at[idx_v.at[j]], rows_s.at[rsl], sem_s))
    for cp in copies:
        cp.wait()
    out_sl = pl.ds(base, _B_PER_W)
    pltpu.sync_copy(rows_t, out_t_hbm.at[out_sl])
    pltpu.sync_copy(rows_s, out_s_hbm.at[out_sl])


def kernel(inputs, timbre_table, speaker_table):
    idx = inputs.astype(jnp.int32).reshape(BATCH // _CHUNK, _CHUNK)
    out_t, out_s = _dual_gather(idx, timbre_table, speaker_table)
    return (out_t, out_s)


# speedup vs baseline: 1.5067x; 1.5067x over previous
import functools

import jax
import jax.numpy as jnp
from jax import lax
from jax.experimental import pallas as pl
from jax.experimental.pallas import tpu as pltpu
from jax.experimental.pallas import tpu_sc as plsc

NUM_EMB = 1000000
EMBEDDING_DIM = 64
BATCH = 16384

_INFO = plsc.get_sparse_core_info()
_NC = _INFO.num_cores
_NS = _INFO.num_subcores
_NW = _NC * _NS
_B_PER_W = BATCH // _NW
_BURST = 16
_NBURST = _B_PER_W // _BURST

_mesh = plsc.VectorSubcoreMesh(core_axis_name="c", subcore_axis_name="s")


@functools.partial(
    pl.kernel,
    mesh=_mesh,
    compiler_params=pltpu.CompilerParams(needs_layout_passes=False),
    out_type=[
        jax.ShapeDtypeStruct((BATCH, EMBEDDING_DIM), jnp.float32),
        jax.ShapeDtypeStruct((BATCH, EMBEDDING_DIM), jnp.float32),
    ],
    scratch_types=[
        pltpu.VMEM((_B_PER_W,), jnp.int32),
        pltpu.VMEM((_B_PER_W, EMBEDDING_DIM), jnp.float32),
        pltpu.SemaphoreType.DMA,
    ],
)
def _dual_gather(idx_hbm, timbre_hbm, speaker_hbm, out_t_hbm, out_s_hbm,
                 idx_v, rows_v, sem):
    wid = lax.axis_index("s") * _NC + lax.axis_index("c")
    base = wid * _B_PER_W
    pltpu.sync_copy(idx_hbm.at[pl.ds(base, _B_PER_W)], idx_v)
    lanes16 = lax.iota(jnp.int32, 16)

    for tbl_hbm, out_hbm in ((timbre_hbm, out_t_hbm), (speaker_hbm, out_s_hbm)):
        def burst_body(b, carry, tbl_hbm=tbl_hbm):
            vec = idx_v[pl.ds(b * _BURST, _BURST)]
            copies = []
            for j in range(_BURST):
                r = jnp.sum(jnp.where(lanes16 == j, vec, 0))
                copies.append(
                    pltpu.async_copy(tbl_hbm.at[r], rows_v.at[b * _BURST + j],
                                     sem))
            for cp in copies:
                cp.wait()
            return carry

        lax.fori_loop(0, _NBURST, burst_body, 0)
        pltpu.sync_copy(rows_v, out_hbm.at[pl.ds(base, _B_PER_W)])


def kernel(inputs, timbre_table, speaker_table):
    idx = inputs.astype(jnp.int32)
    out_t, out_s = _dual_gather(idx, timbre_table, speaker_table)
    return (out_t, out_s)


# speedup vs baseline: 1.5788x; 1.0479x over previous
import functools

import jax
import jax.numpy as jnp
from jax import lax
from jax.experimental import pallas as pl
from jax.experimental.pallas import tpu as pltpu
from jax.experimental.pallas import tpu_sc as plsc

NUM_EMB = 1000000
EMBEDDING_DIM = 64
BATCH = 16384

_INFO = plsc.get_sparse_core_info()
_NC = _INFO.num_cores
_NS = _INFO.num_subcores
_NW = _NC * _NS
_B_PER_W = BATCH // _NW
_HALF = _B_PER_W // 2
_BURST = 16
_NBURST = _HALF // _BURST
_LAG = 2

_mesh = plsc.VectorSubcoreMesh(core_axis_name="c", subcore_axis_name="s")


@functools.partial(
    pl.kernel,
    mesh=_mesh,
    compiler_params=pltpu.CompilerParams(needs_layout_passes=False),
    out_type=[
        jax.ShapeDtypeStruct((BATCH, EMBEDDING_DIM), jnp.float32),
        jax.ShapeDtypeStruct((BATCH, EMBEDDING_DIM), jnp.float32),
    ],
    scratch_types=[
        pltpu.VMEM((_B_PER_W,), jnp.int32),
        pltpu.VMEM((_HALF, EMBEDDING_DIM), jnp.float32),
        pltpu.VMEM((_HALF, EMBEDDING_DIM), jnp.float32),
        pltpu.SemaphoreType.DMA,
        pltpu.SemaphoreType.DMA,
    ],
)
def _dual_gather(idx_hbm, timbre_hbm, speaker_hbm, out_t_hbm, out_s_hbm,
                 idx_v, rows_t, rows_s, sem_t, sem_s):
    wid = lax.axis_index("s") * _NC + lax.axis_index("c")
    base = wid * _B_PER_W
    pltpu.sync_copy(idx_hbm.at[pl.ds(base, _B_PER_W)], idx_v)
    lanes16 = lax.iota(jnp.int32, 16)

    def fire_burst(hoff, b):
        vec = idx_v[pl.ds(hoff + b * _BURST, _BURST)]
        for j in range(_BURST):
            r = jnp.sum(jnp.where(lanes16 == j, vec, 0))
            dst = b * _BURST + j
            pltpu.async_copy(timbre_hbm.at[r], rows_t.at[dst], sem_t)
            pltpu.async_copy(speaker_hbm.at[r], rows_s.at[dst], sem_s)

    def drain_burst():
        for j in range(_BURST):
            pltpu.make_async_copy(timbre_hbm.at[0], rows_t.at[j], sem_t).wait()
            pltpu.make_async_copy(speaker_hbm.at[0], rows_s.at[j], sem_s).wait()

    for half in range(2):
        hoff = half * _HALF
        for b in range(_LAG):
            fire_burst(hoff, b)

        def step(b, carry):
            fire_burst(hoff, b)
            drain_burst()
            return carry

        lax.fori_loop(_LAG, _NBURST, step, 0)
        for _ in range(_LAG):
            drain_burst()
        out_sl = pl.ds(base + hoff, _HALF)
        pltpu.sync_copy(rows_t, out_t_hbm.at[out_sl])
        pltpu.sync_copy(rows_s, out_s_hbm.at[out_sl])


def kernel(inputs, timbre_table, speaker_table):
    idx = inputs.astype(jnp.int32)
    out_t, out_s = _dual_gather(idx, timbre_table, speaker_table)
    return (out_t, out_s)


# speedup vs baseline: 1.5848x; 1.0037x over previous
import functools

import jax
import jax.numpy as jnp
from jax import lax
from jax.experimental import pallas as pl
from jax.experimental.pallas import tpu as pltpu
from jax.experimental.pallas import tpu_sc as plsc

NUM_EMB = 1000000
EMBEDDING_DIM = 64
BATCH = 16384

_INFO = plsc.get_sparse_core_info()
_NC = _INFO.num_cores
_NS = _INFO.num_subcores
_NW = _NC * _NS
_B_PER_W = BATCH // _NW
_HALF = _B_PER_W // 2
_BURST = 16
_NBURST = _HALF // _BURST
_LAG = 6

_mesh = plsc.VectorSubcoreMesh(core_axis_name="c", subcore_axis_name="s")


@functools.partial(
    pl.kernel,
    mesh=_mesh,
    compiler_params=pltpu.CompilerParams(needs_layout_passes=False),
    out_type=[
        jax.ShapeDtypeStruct((BATCH, EMBEDDING_DIM), jnp.float32),
        jax.ShapeDtypeStruct((BATCH, EMBEDDING_DIM), jnp.float32),
    ],
    scratch_types=[
        pltpu.VMEM((_B_PER_W,), jnp.int32),
        pltpu.VMEM((_HALF, EMBEDDING_DIM), jnp.float32),
        pltpu.VMEM((_HALF, EMBEDDING_DIM), jnp.float32),
        pltpu.SemaphoreType.DMA,
        pltpu.SemaphoreType.DMA,
    ],
)
def _dual_gather(idx_hbm, timbre_hbm, speaker_hbm, out_t_hbm, out_s_hbm,
                 idx_v, rows_t, rows_s, sem_t, sem_s):
    wid = lax.axis_index("s") * _NC + lax.axis_index("c")
    base = wid * _B_PER_W
    pltpu.sync_copy(idx_hbm.at[pl.ds(base, _B_PER_W)], idx_v)
    lanes16 = lax.iota(jnp.int32, 16)

    def fire_burst(hoff, b):
        vec = idx_v[pl.ds(hoff + b * _BURST, _BURST)]
        for j in range(_BURST):
            r = jnp.sum(jnp.where(lanes16 == j, vec, 0))
            dst = b * _BURST + j
            pltpu.async_copy(timbre_hbm.at[r], rows_t.at[dst], sem_t)
            pltpu.async_copy(speaker_hbm.at[r], rows_s.at[dst], sem_s)

    def drain_burst():
        bsl = pl.ds(0, _BURST)
        pltpu.make_async_copy(timbre_hbm.at[bsl], rows_t.at[bsl], sem_t).wait()
        pltpu.make_async_copy(speaker_hbm.at[bsl], rows_s.at[bsl], sem_s).wait()

    for half in range(2):
        hoff = half * _HALF
        for b in range(_LAG):
            fire_burst(hoff, b)

        def step(b, carry):
            fire_burst(hoff, b)
            drain_burst()
            return carry

        lax.fori_loop(_LAG, _NBURST, step, 0)
        for _ in range(_LAG):
            drain_burst()
        out_sl = pl.ds(base + hoff, _HALF)
        pltpu.sync_copy(rows_t, out_t_hbm.at[out_sl])
        pltpu.sync_copy(rows_s, out_s_hbm.at[out_sl])


def kernel(inputs, timbre_table, speaker_table):
    idx = inputs.astype(jnp.int32)
    out_t, out_s = _dual_gather(idx, timbre_table, speaker_table)
    return (out_t, out_s)
